# baseline (device time: 19816 ns/iter reference)
import jax
import jax.numpy as jnp
from jax import lax
from jax.experimental import pallas as pl
from jax.experimental.pallas import tpu as pltpu

N_Z = 4
CHUNK_ROWS = (64, 192, 192, 64)
K = len(CHUNK_ROWS)
CHUNK_OFF = tuple(sum(CHUNK_ROWS[:i]) for i in range(K))

E2M, M2M, M2E = 0, 1, 2


def kernel(x):
    m, n = x.shape
    assert sum(CHUNK_ROWS) == m

    def body(x_ref, out_ref, in_bf, peer_end, pairsum, peer_mid,
             send_sems, recv_sems):
        my_x = lax.axis_index("x")
        my_y = lax.axis_index("y")
        my_z = lax.axis_index("z")
        is_end = jnp.logical_or(my_z == 0, my_z == N_Z - 1)

        in_bf[...] = x_ref[...].astype(jnp.bfloat16)

        def chunk(ref, i):
            return ref.at[pl.ds(CHUNK_OFF[i], CHUNK_ROWS[i]), :]

        def desc(src, dst, phase, i, dev_z):
            return pltpu.make_async_remote_copy(
                src_ref=src,
                dst_ref=dst,
                send_sem=send_sems.at[phase, i],
                recv_sem=recv_sems.at[phase, i],
                device_id=(my_x, my_y, dev_z),
                device_id_type=pl.DeviceIdType.MESH,
            )

        barrier_sem = pltpu.get_barrier_semaphore()

        @pl.when(is_end)
        def _():
            mid = jnp.where(my_z == 0, 1, 2)
            pl.semaphore_signal(
                barrier_sem, inc=1,
                device_id=(my_x, my_y, mid),
                device_id_type=pl.DeviceIdType.MESH,
            )
            pl.semaphore_wait(barrier_sem, 1)

            e2m = [
                desc(chunk(in_bf, i), chunk(peer_end, i), E2M, i, mid)
                for i in range(K)
            ]
            for i in range(K):
                e2m[i].start()
            for i in range(K):
                r = desc(chunk(out_ref, i), chunk(out_ref, i), M2E, i, mid)
                r.wait_recv()
            for i in range(K):
                e2m[i].wait_send()

        @pl.when(jnp.logical_not(is_end))
        def _():
            end = jnp.where(my_z == 1, 0, 3)
            other = N_Z - 1 - my_z
            for nbr in (end, other):
                pl.semaphore_signal(
                    barrier_sem, inc=1,
                    device_id=(my_x, my_y, nbr),
                    device_id_type=pl.DeviceIdType.MESH,
                )
            pl.semaphore_wait(barrier_sem, 2)

            def finish_chunk(j):
                r = desc(chunk(pairsum, j), chunk(peer_mid, j), M2M, j, other)
                r.wait_recv()
                chunk(out_ref, j)[...] = (
                    chunk(pairsum, j)[...] + chunk(peer_mid, j)[...])
                s = desc(chunk(out_ref, j), chunk(out_ref, j), M2E, j, end)
                s.start()
                return s

            sends = []
            for i in range(K):
                r = desc(chunk(in_bf, i), chunk(peer_end, i), E2M, i, end)
                r.wait_recv()
                chunk(pairsum, i)[...] = (
                    chunk(in_bf, i)[...] + chunk(peer_end, i)[...])
                s = desc(chunk(pairsum, i), chunk(peer_mid, i), M2M, i, other)
                s.start()
                sends.append(s)
                if i >= 1:
                    sends.append(finish_chunk(i - 1))
            sends.append(finish_chunk(K - 1))
            for s in sends:
                s.wait_send()

    return pl.pallas_call(
        body,
        out_shape=jax.ShapeDtypeStruct((m, n), jnp.bfloat16),
        in_specs=[pl.BlockSpec(memory_space=pltpu.VMEM)],
        out_specs=pl.BlockSpec(memory_space=pltpu.VMEM),
        scratch_shapes=[
            pltpu.VMEM((m, n), jnp.bfloat16),
            pltpu.VMEM((m, n), jnp.bfloat16),
            pltpu.VMEM((m, n), jnp.bfloat16),
            pltpu.VMEM((m, n), jnp.bfloat16),
            pltpu.SemaphoreType.DMA((3, K)),
            pltpu.SemaphoreType.DMA((3, K)),
        ],
        compiler_params=pltpu.CompilerParams(collective_id=0),
    )(x)


# device time: 18643 ns/iter; 1.0629x vs baseline; 1.0629x over previous
import jax
import jax.numpy as jnp
from jax import lax
from jax.experimental import pallas as pl
from jax.experimental.pallas import tpu as pltpu

N_Z = 4
CHUNK_ROWS = (128, 128, 128, 128)
K = len(CHUNK_ROWS)
CHUNK_OFF = tuple(sum(CHUNK_ROWS[:i]) for i in range(K))

E2M, M2M, M2E = 0, 1, 2


def kernel(x):
    m, n = x.shape
    assert sum(CHUNK_ROWS) == m

    def body(x_ref, out_ref, in_bf, peer_end, pairsum, peer_mid,
             send_sems, recv_sems):
        my_x = lax.axis_index("x")
        my_y = lax.axis_index("y")
        my_z = lax.axis_index("z")
        is_end = jnp.logical_or(my_z == 0, my_z == N_Z - 1)

        in_bf[...] = x_ref[...].astype(jnp.bfloat16)

        def chunk(ref, i):
            return ref.at[pl.ds(CHUNK_OFF[i], CHUNK_ROWS[i]), :]

        def desc(src, dst, phase, i, dev_z):
            return pltpu.make_async_remote_copy(
                src_ref=src,
                dst_ref=dst,
                send_sem=send_sems.at[phase, i],
                recv_sem=recv_sems.at[phase, i],
                device_id=(my_x, my_y, dev_z),
                device_id_type=pl.DeviceIdType.MESH,
            )

        barrier_sem = pltpu.get_barrier_semaphore()

        @pl.when(is_end)
        def _():
            mid = jnp.where(my_z == 0, 1, 2)
            pl.semaphore_signal(
                barrier_sem, inc=1,
                device_id=(my_x, my_y, mid),
                device_id_type=pl.DeviceIdType.MESH,
            )
            pl.semaphore_wait(barrier_sem, 1)

            e2m = [
                desc(chunk(in_bf, i), chunk(peer_end, i), E2M, i, mid)
                for i in range(K)
            ]
            for i in range(K):
                e2m[i].start()
            for i in range(K):
                r = desc(chunk(out_ref, i), chunk(out_ref, i), M2E, i, mid)
                r.wait_recv()
            for i in range(K):
                e2m[i].wait_send()

        @pl.when(jnp.logical_not(is_end))
        def _():
            end = jnp.where(my_z == 1, 0, 3)
            other = N_Z - 1 - my_z
            for nbr in (end, other):
                pl.semaphore_signal(
                    barrier_sem, inc=1,
                    device_id=(my_x, my_y, nbr),
                    device_id_type=pl.DeviceIdType.MESH,
                )
            pl.semaphore_wait(barrier_sem, 2)

            def finish_chunk(j):
                r = desc(chunk(pairsum, j), chunk(peer_mid, j), M2M, j, other)
                r.wait_recv()
                chunk(out_ref, j)[...] = (
                    chunk(pairsum, j)[...] + chunk(peer_mid, j)[...])
                s = desc(chunk(out_ref, j), chunk(out_ref, j), M2E, j, end)
                s.start()
                return s

            sends = []
            for i in range(K):
                r = desc(chunk(in_bf, i), chunk(peer_end, i), E2M, i, end)
                r.wait_recv()
                chunk(pairsum, i)[...] = (
                    chunk(in_bf, i)[...] + chunk(peer_end, i)[...])
                s = desc(chunk(pairsum, i), chunk(peer_mid, i), M2M, i, other)
                s.start()
                sends.append(s)
                if i >= 1:
                    sends.append(finish_chunk(i - 1))
            sends.append(finish_chunk(K - 1))
            for s in sends:
                s.wait_send()

    return pl.pallas_call(
        body,
        out_shape=jax.ShapeDtypeStruct((m, n), jnp.bfloat16),
        in_specs=[pl.BlockSpec(memory_space=pltpu.VMEM)],
        out_specs=pl.BlockSpec(memory_space=pltpu.VMEM),
        scratch_shapes=[
            pltpu.VMEM((m, n), jnp.bfloat16),
            pltpu.VMEM((m, n), jnp.bfloat16),
            pltpu.VMEM((m, n), jnp.bfloat16),
            pltpu.VMEM((m, n), jnp.bfloat16),
            pltpu.SemaphoreType.DMA((3, K)),
            pltpu.SemaphoreType.DMA((3, K)),
        ],
        compiler_params=pltpu.CompilerParams(collective_id=0),
    )(x)
